# baseline (speedup 1.0000x reference)
import functools

import jax
import jax.numpy as jnp
from jax import lax
from jax.experimental import pallas as pl
from jax.experimental.pallas import tpu as pltpu
from jax.experimental.pallas import tpu_sc as plsc

N_POI = 10000
HID = 256
N_EDGES = 160000
HALF = 128
NTILE = 16
G = 64
NG = 165
EPT = NG * G
EPAD = EPT * NTILE
APAD = 10240
RPT = APAD // NTILE
NBUF = 3
NEG_SLOPE = 0.01


def _make_propagate():
    mesh = plsc.VectorSubcoreMesh(core_axis_name="c", subcore_axis_name="s")

    @functools.partial(
        pl.kernel,
        mesh=mesh,
        compiler_params=pltpu.CompilerParams(use_tc_tiling_on_sc=False),
        out_type=jax.ShapeDtypeStruct((2, APAD, HALF), jnp.float32),
        scratch_types=[
            pltpu.VMEM_SHARED((APAD, HALF), jnp.float32),
            pltpu.VMEM((NG, G), jnp.int32),
            pltpu.VMEM((NG, G), jnp.int32),
            [pltpu.VMEM((G, HALF), jnp.float32) for _ in range(NBUF)],
            [pltpu.VMEM((G // 8, 128), jnp.float32) for _ in range(NBUF)],
            [pltpu.SemaphoreType.DMA for _ in range(NBUF)],
            [pltpu.SemaphoreType.DMA for _ in range(NBUF)],
            [pltpu.SemaphoreType.DMA for _ in range(NBUF)],
        ],
    )
    def propagate(x_hbm, src_hbm, dst_hbm, dist_hbm, z_hbm, out_hbm,
                  acc, idxb, dstb, gbufs, dbufs, gsems, dsems, ssems):
        c = lax.axis_index("c")
        s = lax.axis_index("s")

        pltpu.sync_copy(src_hbm.at[s], idxb)
        pltpu.sync_copy(dst_hbm.at[s], dstb)

        def idx_body(g, carry):
            for k in range(G // 16):
                sl = pl.ds(k * 16, 16)
                idxb[g, sl] = idxb[g, sl] * 2 + c
            return carry
        lax.fori_loop(0, NG, idx_body, 0)

        pltpu.sync_copy(z_hbm, acc.at[pl.ds(s * RPT, RPT)])
        plsc.subcore_barrier()

        def start_group(g, b):
            pltpu.make_async_copy(x_hbm.at[idxb.at[g]], gbufs[b], gsems[b]).start()
            pltpu.make_async_copy(dist_hbm.at[s, g], dbufs[b], dsems[b]).start()

        def process_group(g, b):
            pltpu.make_async_copy(x_hbm.at[idxb.at[g]], gbufs[b], gsems[b]).wait()
            pltpu.make_async_copy(dist_hbm.at[s, g], dbufs[b], dsems[b]).wait()
            gb = gbufs[b]
            db = dbufs[b]
            for e in range(G):
                dv = db[e // 8, pl.ds((e % 8) * 16, 16)]
                w = jnp.exp(-(dv * dv))
                for j in range(HALF // 16):
                    sl = pl.ds(j * 16, 16)
                    gb[e, sl] = gb[e, sl] * w
            pltpu.sync_copy(gb, acc.at[dstb.at[g]], add=True)

        def wait_scatter(g, b):
            del g, b

        for b in range(NBUF):
            start_group(b, b)

        def loop_body(i, carry):
            g0 = i * NBUF
            for k in range(NBUF):
                g = g0 + k
                process_group(g, k)
                prev = g - 2
                pb = (k - 2) % NBUF

                @pl.when(prev >= 0)
                def _():
                    wait_scatter(prev, pb)

                    @pl.when(prev + NBUF < NG)
                    def _():
                        start_group(prev + NBUF, pb)
            return carry
        lax.fori_loop(0, NG // NBUF, loop_body, 0)

        wait_scatter(NG - 2, (NG - 2) % NBUF)
        wait_scatter(NG - 1, (NG - 1) % NBUF)
        plsc.subcore_barrier()
        pltpu.sync_copy(acc.at[pl.ds(s * RPT, RPT)],
                        out_hbm.at[c, pl.ds(s * RPT, RPT)])

    return propagate


_propagate = _make_propagate()

_TC_ROWS = 1000


def _tc_layer(S, X, W, b, acc, *, scale):
    def body(s_ref, x_ref, w_ref, b_ref, a_ref, y_ref, aout_ref):
        srow = jnp.concatenate([s_ref[0], s_ref[1]], axis=-1)
        h = srow + x_ref[...]
        z = jnp.dot(h, w_ref[...], preferred_element_type=jnp.float32) + b_ref[...]
        y = jnp.where(z >= 0, z, NEG_SLOPE * z)
        y_ref[...] = y
        aout_ref[...] = (a_ref[...] + y) * scale

    return pl.pallas_call(
        body,
        grid=(N_POI // _TC_ROWS,),
        in_specs=[
            pl.BlockSpec((2, _TC_ROWS, HALF), lambda i: (0, i, 0)),
            pl.BlockSpec((_TC_ROWS, HID), lambda i: (i, 0)),
            pl.BlockSpec((HID, HID), lambda i: (0, 0)),
            pl.BlockSpec((1, HID), lambda i: (0, 0)),
            pl.BlockSpec((_TC_ROWS, HID), lambda i: (i, 0)),
        ],
        out_specs=[
            pl.BlockSpec((_TC_ROWS, HID), lambda i: (i, 0)),
            pl.BlockSpec((_TC_ROWS, HID), lambda i: (i, 0)),
        ],
        out_shape=[
            jax.ShapeDtypeStruct((N_POI, HID), jnp.float32),
            jax.ShapeDtypeStruct((N_POI, HID), jnp.float32),
        ],
    )(S, X, W, b, acc)


def kernel(poi_embs, edge_index, dist, W0, b0, W1, b1, W2, b2):
    src = edge_index[0].astype(jnp.int32)
    dst = edge_index[1].astype(jnp.int32)
    pad = EPAD - N_EDGES
    src_p = jnp.concatenate([src, jnp.zeros((pad,), jnp.int32)]).reshape(NTILE, NG, G)
    dst_p = jnp.concatenate([dst, jnp.zeros((pad,), jnp.int32)]).reshape(NTILE, NG, G)
    dist_p = jnp.concatenate(
        [dist.astype(jnp.float32), jnp.full((pad,), 30.0, jnp.float32)])
    dist_e = jnp.broadcast_to(dist_p[:, None], (EPAD, 16)).reshape(
        NTILE, NG, G // 8, 128)
    zeros = jnp.zeros((RPT, HALF), jnp.float32)

    X = poi_embs
    acc = X
    for l, (W, b) in enumerate(((W0, b0), (W1, b1), (W2, b2))):
        S = _propagate(X.reshape(2 * N_POI, HALF), src_p, dst_p, dist_e, zeros)
        X, acc = _tc_layer(S, X, W, b.reshape(1, HID), acc,
                           scale=(0.25 if l == 2 else 1.0))
    return acc


# speedup vs baseline: 2.1592x; 2.1592x over previous
import functools

import jax
import jax.numpy as jnp
from jax import lax
from jax.experimental import pallas as pl
from jax.experimental.pallas import tpu as pltpu
from jax.experimental.pallas import tpu_sc as plsc

N_POI = 10000
HID = 256
N_EDGES = 160000
HALF = 128
NTILE = 16
G = 80
NG = 128
EPT = NG * G
EPAD = EPT * NTILE
APAD = 10240
RPT = APAD // NTILE
NBUF = 2
NEG_SLOPE = 0.01


def _make_propagate():
    mesh = plsc.VectorSubcoreMesh(core_axis_name="c", subcore_axis_name="s")

    @functools.partial(
        pl.kernel,
        mesh=mesh,
        compiler_params=pltpu.CompilerParams(use_tc_tiling_on_sc=False),
        out_type=jax.ShapeDtypeStruct((2, APAD, HALF), jnp.float32),
        scratch_types=[
            pltpu.VMEM_SHARED((APAD, HALF), jnp.float32),
            pltpu.VMEM((NG, G), jnp.int32),
            pltpu.VMEM((NG, G), jnp.int32),
            [pltpu.VMEM((G, HALF), jnp.float32) for _ in range(NBUF)],
            [pltpu.VMEM((G // 8, 128), jnp.float32) for _ in range(NBUF)],
            [pltpu.SemaphoreType.DMA for _ in range(NBUF)],
            [pltpu.SemaphoreType.DMA for _ in range(NBUF)],
            [pltpu.SemaphoreType.DMA for _ in range(NBUF)],
        ],
    )
    def propagate(x_hbm, src_hbm, dst_hbm, dist_hbm, z_hbm, out_hbm,
                  acc, idxb, dstb, gbufs, dbufs, gsems, dsems, ssems):
        c = lax.axis_index("c")
        s = lax.axis_index("s")

        pltpu.sync_copy(src_hbm.at[s], idxb)
        pltpu.sync_copy(dst_hbm.at[s], dstb)

        def idx_body(g, carry):
            for k in range(G // 16):
                sl = pl.ds(k * 16, 16)
                idxb[g, sl] = idxb[g, sl] * 2 + c
            return carry
        lax.fori_loop(0, NG, idx_body, 0)

        pltpu.sync_copy(z_hbm, acc.at[pl.ds(s * RPT, RPT)])
        plsc.subcore_barrier()

        def start_group(g, b):
            pltpu.make_async_copy(x_hbm.at[idxb.at[g]], gbufs[b], gsems[b]).start()
            pltpu.make_async_copy(dist_hbm.at[s, g], dbufs[b], dsems[b]).start()

        def process_group(g, b):
            pltpu.make_async_copy(x_hbm.at[idxb.at[g]], gbufs[b], gsems[b]).wait()
            pltpu.make_async_copy(dist_hbm.at[s, g], dbufs[b], dsems[b]).wait()
            gb = gbufs[b]
            db = dbufs[b]
            for e in range(G):
                dv = db[e // 8, pl.ds((e % 8) * 16, 16)]
                w = jnp.exp(-(dv * dv))
                for j in range(HALF // 16):
                    sl = pl.ds(j * 16, 16)
                    gb[e, sl] = gb[e, sl] * w
            pltpu.sync_copy(gb, acc.at[dstb.at[g]], add=True)

        def wait_scatter(g, b):
            del g, b

        for b in range(NBUF):
            start_group(b, b)

        def loop_body(i, carry):
            g0 = i * NBUF
            for k in range(NBUF):
                g = g0 + k
                process_group(g, k)

                @pl.when(g + NBUF < NG)
                def _():
                    start_group(g + NBUF, k)
            return carry
        lax.fori_loop(0, NG // NBUF, loop_body, 0)

        plsc.subcore_barrier()
        pltpu.sync_copy(acc.at[pl.ds(s * RPT, RPT)],
                        out_hbm.at[c, pl.ds(s * RPT, RPT)])

    return propagate


_propagate = _make_propagate()

_TC_ROWS = 1000


def _tc_layer(S, X, W, b, acc, *, scale):
    def body(s_ref, x_ref, w_ref, b_ref, a_ref, y_ref, aout_ref):
        srow = jnp.concatenate([s_ref[0], s_ref[1]], axis=-1)
        h = srow + x_ref[...]
        z = jnp.dot(h, w_ref[...], preferred_element_type=jnp.float32) + b_ref[...]
        y = jnp.where(z >= 0, z, NEG_SLOPE * z)
        y_ref[...] = y
        aout_ref[...] = (a_ref[...] + y) * scale

    return pl.pallas_call(
        body,
        grid=(N_POI // _TC_ROWS,),
        in_specs=[
            pl.BlockSpec((2, _TC_ROWS, HALF), lambda i: (0, i, 0)),
            pl.BlockSpec((_TC_ROWS, HID), lambda i: (i, 0)),
            pl.BlockSpec((HID, HID), lambda i: (0, 0)),
            pl.BlockSpec((1, HID), lambda i: (0, 0)),
            pl.BlockSpec((_TC_ROWS, HID), lambda i: (i, 0)),
        ],
        out_specs=[
            pl.BlockSpec((_TC_ROWS, HID), lambda i: (i, 0)),
            pl.BlockSpec((_TC_ROWS, HID), lambda i: (i, 0)),
        ],
        out_shape=[
            jax.ShapeDtypeStruct((N_POI, HID), jnp.float32),
            jax.ShapeDtypeStruct((N_POI, HID), jnp.float32),
        ],
    )(S, X, W, b, acc)


def kernel(poi_embs, edge_index, dist, W0, b0, W1, b1, W2, b2):
    src = edge_index[0].astype(jnp.int32)
    dst = edge_index[1].astype(jnp.int32)
    pad = EPAD - N_EDGES
    src_p = jnp.concatenate([src, jnp.zeros((pad,), jnp.int32)]).reshape(NTILE, NG, G)
    dst_p = jnp.concatenate([dst, jnp.zeros((pad,), jnp.int32)]).reshape(NTILE, NG, G)
    dist_p = jnp.concatenate(
        [dist.astype(jnp.float32), jnp.full((pad,), 30.0, jnp.float32)])
    dist_e = jnp.broadcast_to(dist_p[:, None], (EPAD, 16)).reshape(
        NTILE, NG, G // 8, 128)
    zeros = jnp.zeros((RPT, HALF), jnp.float32)

    X = poi_embs
    acc = X
    for l, (W, b) in enumerate(((W0, b0), (W1, b1), (W2, b2))):
        S = _propagate(X.reshape(2 * N_POI, HALF), src_p, dst_p, dist_e, zeros)
        X, acc = _tc_layer(S, X, W, b.reshape(1, HID), acc,
                           scale=(0.25 if l == 2 else 1.0))
    return acc
